# baseline (speedup 1.0000x reference)
import functools

import jax
import jax.numpy as jnp
from jax import lax
from jax.experimental import pallas as pl
from jax.experimental.pallas import tpu as pltpu
from jax.experimental.pallas import tpu_sc as plsc

FEAT = 512
CDIM = 256
K = 1024
B = 32
T = 128
NTOK = B * T
CHUNK = 1024
NCH = 2
BLK = CHUNK * NCH
NBLK = NTOK // BLK
CPB = CHUNK // T

_DN = (((1,), (1,)), ((), ()))


def _ln(h, g, b):
    m = jnp.mean(h, axis=-1, keepdims=True)
    v = jnp.mean((h - m) ** 2, axis=-1, keepdims=True)
    return (h - m) / jnp.sqrt(v + 1e-5) * g + b


def _row(ref):
    return ref[...].reshape(1, -1)


def _enc_body(x_ref, eW1, eb1, g1, b1, eW2, eb2, g2, b2, cb,
              idx_out, commit_out, ct_sc, cbsq_sc, cacc):
    i = pl.program_id(0)

    @pl.when(i == 0)
    def _():
        ct0 = cb[...].T
        ct_sc[...] = ct0
        cbsq_sc[...] = jnp.sum(ct0 * ct0, axis=0, keepdims=True)

    ct = ct_sc[...]
    cbsq = cbsq_sc[...]
    cs = 0.0
    for c in range(NCH):
        xb = x_ref[c * CHUNK:(c + 1) * CHUNK, :]
        h = jnp.dot(xb, eW1[...], preferred_element_type=jnp.float32) + _row(eb1)
        h = jnp.maximum(_ln(h, _row(g1), _row(b1)), 0.0)
        h = jnp.dot(h, eW2[...], preferred_element_type=jnp.float32) + _row(eb2)
        enc = jnp.maximum(_ln(h, _row(g2), _row(b2)), 0.0)

        s2 = jnp.dot(enc * -2.0, ct, preferred_element_type=jnp.float32)
        d2 = s2 + cbsq
        mins = jnp.min(d2, axis=1, keepdims=True)
        col = lax.broadcasted_iota(jnp.int32, (CHUNK, K), 1)
        idx = jnp.min(jnp.where(d2 <= mins, col, K), axis=1, keepdims=True)
        idx_out[c * CHUNK:(c + 1) * CHUNK, :] = idx

        cs += jnp.sum(enc * enc) + jnp.sum(mins)

    @pl.when(i == 0)
    def _():
        cacc[0, 0] = cs

    @pl.when(i > 0)
    def _():
        cacc[0, 0] += cs

    @pl.when(i == NBLK - 1)
    def _():
        commit_out[...] = jnp.broadcast_to(cacc[0, 0] / (NTOK * CDIM), (1, 1))


def _dec_body(x_ref, q_ref, dW1, db1, dW2, db2, pW1, pb1, pW2, pb2,
              commit_in,
              recon_out, commit_out, cbl_out, contr_out, total_out,
              pooled_sc, racc):
    i = pl.program_id(0)
    rs = 0.0
    for c in range(NCH):
        xb = x_ref[c * CHUNK:(c + 1) * CHUNK, :]
        q = q_ref[c * CHUNK:(c + 1) * CHUNK, :]
        hd = jnp.maximum(
            jnp.dot(q, dW1[...], preferred_element_type=jnp.float32)
            + _row(db1), 0.0)
        r = jnp.dot(hd, dW2[...], preferred_element_type=jnp.float32) + _row(db2)
        rs += jnp.sum((r - xb) ** 2)

        rid = lax.broadcasted_iota(jnp.int32, (CPB, CHUNK), 0)
        cid = lax.broadcasted_iota(jnp.int32, (CPB, CHUNK), 1)
        M = jnp.where(cid // T == rid, 1.0 / T, 0.0)
        pooled_sc[pl.ds(i * (NCH * CPB) + c * CPB, CPB), :] = jnp.dot(
            M, q, preferred_element_type=jnp.float32)

    @pl.when(i == 0)
    def _():
        racc[0, 0] = rs

    @pl.when(i > 0)
    def _():
        racc[0, 0] += rs

    @pl.when(i == NBLK - 1)
    def _():
        pooled = pooled_sc[...]
        p = jnp.maximum(
            jnp.dot(pooled, pW1[...], preferred_element_type=jnp.float32)
            + _row(pb1), 0.0)
        p = jnp.dot(p, pW2[...], preferred_element_type=jnp.float32) + _row(pb2)
        nrm = jnp.maximum(jnp.sqrt(jnp.sum(p * p, axis=1, keepdims=True)),
                          1e-12)
        n = p / nrm
        sim = lax.dot_general(n, n, _DN,
                              preferred_element_type=jnp.float32) / 0.1
        mx = jnp.max(sim, axis=1, keepdims=True)
        logp = sim - mx - jnp.log(jnp.sum(jnp.exp(sim - mx), axis=1,
                                          keepdims=True))
        er = lax.broadcasted_iota(jnp.int32, (B, B), 0)
        ec = lax.broadcasted_iota(jnp.int32, (B, B), 1)
        contr = -jnp.sum(jnp.where(er == ec, logp, 0.0)) / B
        recon = racc[0, 0] / (NTOK * FEAT)
        commit = commit_in[0, 0]
        w = lambda ref, v: ref.__setitem__(
            (slice(None), slice(None)), jnp.broadcast_to(v, (1, 1)))
        w(recon_out, recon)
        w(commit_out, commit)
        w(cbl_out, commit)
        w(contr_out, contr)
        w(total_out, recon + commit * 0.25 + commit + contr * 0.1)


def _sc_gather(codebook, idx_flat):
    info = plsc.get_sparse_core_info()
    nw = info.num_cores * info.num_subcores
    bpw = NTOK // nw
    mesh = plsc.VectorSubcoreMesh(core_axis_name="c", subcore_axis_name="s")

    @functools.partial(
        pl.kernel, mesh=mesh,
        out_type=jax.ShapeDtypeStruct((NTOK, CDIM), jnp.float32),
        scratch_types=[
            pltpu.VMEM((bpw,), jnp.int32),
            pltpu.VMEM((bpw, CDIM), jnp.float32),
            pltpu.SemaphoreType.DMA,
        ],
    )
    def k(table_hbm, idx_hbm, out_hbm, idx_v, rows_v, sem):
        wid = lax.axis_index("s") * info.num_cores + lax.axis_index("c")
        base = wid * bpw
        pltpu.sync_copy(idx_hbm.at[pl.ds(base, bpw)], idx_v)
        pltpu.async_copy(table_hbm.at[idx_v], rows_v, sem).wait()
        pltpu.sync_copy(rows_v, out_hbm.at[pl.ds(base, bpw)])

    return k(codebook, idx_flat)


def kernel(x, enc_W1, enc_b1, ln1_g, ln1_b, enc_W2, enc_b2, ln2_g, ln2_b,
           codebook, dec_W1, dec_b1, dec_W2, dec_b2, proj_W1, proj_b1,
           proj_W2, proj_b2):
    xf = x.reshape(NTOK, FEAT)
    full = lambda a: pl.BlockSpec(a.shape, lambda i: (0,) * a.ndim)
    sc = pl.BlockSpec((1, 1), lambda i: (0, 0))

    enc_args = (xf, enc_W1, enc_b1, ln1_g, ln1_b, enc_W2, enc_b2,
                ln2_g, ln2_b, codebook)
    idx, commit = pl.pallas_call(
        _enc_body,
        grid=(NBLK,),
        in_specs=[pl.BlockSpec((BLK, FEAT), lambda i: (i, 0))] + [
            full(a) for a in enc_args[1:]],
        out_specs=[pl.BlockSpec((BLK, 1), lambda i: (i, 0)), sc],
        out_shape=[jax.ShapeDtypeStruct((NTOK, 1), jnp.int32),
                   jax.ShapeDtypeStruct((1, 1), jnp.float32)],
        scratch_shapes=[pltpu.VMEM((CDIM, K), jnp.float32),
                        pltpu.VMEM((1, K), jnp.float32),
                        pltpu.SMEM((1, 1), jnp.float32)],
    )(*enc_args)

    q = _sc_gather(codebook, idx.reshape(NTOK))

    dec_args = (xf, q, dec_W1, dec_b1, dec_W2, dec_b2,
                proj_W1, proj_b1, proj_W2, proj_b2, commit)
    out = pl.pallas_call(
        _dec_body,
        grid=(NBLK,),
        in_specs=[pl.BlockSpec((BLK, FEAT), lambda i: (i, 0)),
                  pl.BlockSpec((BLK, CDIM), lambda i: (i, 0))] + [
            full(a) for a in dec_args[2:10]] + [sc],
        out_specs=[sc] * 5,
        out_shape=[jax.ShapeDtypeStruct((1, 1), jnp.float32)] * 5,
        scratch_shapes=[pltpu.VMEM((B, CDIM), jnp.float32),
                        pltpu.SMEM((1, 1), jnp.float32)],
    )(*dec_args)
    recon, commit_o, cbl, contr, total = out
    return (idx.reshape(B, T), recon[0, 0], commit_o[0, 0], cbl[0, 0],
            contr[0, 0], total[0, 0])


# speedup vs baseline: 1.7992x; 1.7992x over previous
import jax
import jax.numpy as jnp
from jax import lax
from jax.experimental import pallas as pl
from jax.experimental.pallas import tpu as pltpu

FEAT = 512
CDIM = 256
K = 1024
B = 32
T = 128
NTOK = B * T
CHUNK = 1024
NCH = 2
BLK = CHUNK * NCH
NBLK = NTOK // BLK
CPB = CHUNK // T

_DN = (((1,), (1,)), ((), ()))


def _ln(h, g, b):
    m = jnp.mean(h, axis=-1, keepdims=True)
    v = jnp.mean((h - m) ** 2, axis=-1, keepdims=True)
    return (h - m) / jnp.sqrt(v + 1e-5) * g + b


def _row(ref):
    return ref[...].reshape(1, -1)


def _body(x_ref, eW1, eb1, g1, b1, eW2, eb2, g2, b2, cb,
          dW1, db1, dW2, db2, pW1, pb1, pW2, pb2,
          idx_out, recon_out, commit_out, cbl_out, contr_out, total_out,
          pooled_sc, ct_sc, cbsq_sc, racc, cacc):
    i = pl.program_id(0)
    cbv = cb[...]

    @pl.when(i == 0)
    def _():
        ct0 = cbv.T
        ct_sc[...] = ct0
        cbsq_sc[...] = jnp.sum(ct0 * ct0, axis=0, keepdims=True)

    ct = ct_sc[...]
    cbsq = cbsq_sc[...]
    rs = 0.0
    cs = 0.0
    for c in range(NCH):
        xb = x_ref[c * CHUNK:(c + 1) * CHUNK, :]
        h = jnp.dot(xb, eW1[...], preferred_element_type=jnp.float32) + _row(eb1)
        h = jnp.maximum(_ln(h, _row(g1), _row(b1)), 0.0)
        h = jnp.dot(h, eW2[...], preferred_element_type=jnp.float32) + _row(eb2)
        enc = jnp.maximum(_ln(h, _row(g2), _row(b2)), 0.0)

        s2 = jnp.dot(enc * -2.0, ct, preferred_element_type=jnp.float32)
        d2 = s2 + cbsq
        mins = jnp.min(d2, axis=1, keepdims=True)
        col = lax.broadcasted_iota(jnp.int32, (CHUNK, K), 1)
        idx = jnp.min(jnp.where(d2 <= mins, col, K), axis=1, keepdims=True)
        idx_out[c * CHUNK:(c + 1) * CHUNK, :] = idx

        cs += jnp.sum(enc * enc) + jnp.sum(mins)

        onehot = jnp.where(col == idx, 1.0, 0.0)
        q = jnp.dot(onehot, cbv, preferred_element_type=jnp.float32)

        hd = jnp.maximum(
            jnp.dot(q, dW1[...], preferred_element_type=jnp.float32)
            + _row(db1), 0.0)
        r = jnp.dot(hd, dW2[...], preferred_element_type=jnp.float32) + _row(db2)
        rs += jnp.sum((r - xb) ** 2)

        rid = lax.broadcasted_iota(jnp.int32, (CPB, CHUNK), 0)
        cid = lax.broadcasted_iota(jnp.int32, (CPB, CHUNK), 1)
        M = jnp.where(cid // T == rid, 1.0 / T, 0.0)
        pooled_sc[pl.ds(i * (NCH * CPB) + c * CPB, CPB), :] = jnp.dot(
            M, q, preferred_element_type=jnp.float32)

    @pl.when(i == 0)
    def _():
        racc[0, 0] = rs
        cacc[0, 0] = cs

    @pl.when(i > 0)
    def _():
        racc[0, 0] += rs
        cacc[0, 0] += cs

    @pl.when(i == NBLK - 1)
    def _():
        pooled = pooled_sc[...]
        p = jnp.maximum(
            jnp.dot(pooled, pW1[...], preferred_element_type=jnp.float32)
            + _row(pb1), 0.0)
        p = jnp.dot(p, pW2[...], preferred_element_type=jnp.float32) + _row(pb2)
        nrm = jnp.maximum(jnp.sqrt(jnp.sum(p * p, axis=1, keepdims=True)),
                          1e-12)
        n = p / nrm
        sim = lax.dot_general(n, n, _DN,
                              preferred_element_type=jnp.float32) / 0.1
        mx = jnp.max(sim, axis=1, keepdims=True)
        logp = sim - mx - jnp.log(jnp.sum(jnp.exp(sim - mx), axis=1,
                                          keepdims=True))
        er = lax.broadcasted_iota(jnp.int32, (B, B), 0)
        ec = lax.broadcasted_iota(jnp.int32, (B, B), 1)
        contr = -jnp.sum(jnp.where(er == ec, logp, 0.0)) / B
        recon = racc[0, 0] / (NTOK * FEAT)
        commit = cacc[0, 0] / (NTOK * CDIM)
        w = lambda ref, v: ref.__setitem__(
            (slice(None), slice(None)), jnp.broadcast_to(v, (1, 1)))
        w(recon_out, recon)
        w(commit_out, commit)
        w(cbl_out, commit)
        w(contr_out, contr)
        w(total_out, recon + commit * 0.25 + commit + contr * 0.1)


def kernel(x, enc_W1, enc_b1, ln1_g, ln1_b, enc_W2, enc_b2, ln2_g, ln2_b,
           codebook, dec_W1, dec_b1, dec_W2, dec_b2, proj_W1, proj_b1,
           proj_W2, proj_b2):
    xf = x.reshape(NTOK, FEAT)
    full = lambda a: pl.BlockSpec(a.shape, lambda i: (0,) * a.ndim)
    args = (xf, enc_W1, enc_b1, ln1_g, ln1_b, enc_W2, enc_b2,
            ln2_g, ln2_b, codebook, dec_W1, dec_b1,
            dec_W2, dec_b2, proj_W1, proj_b1, proj_W2, proj_b2)
    in_specs = [pl.BlockSpec((BLK, FEAT), lambda i: (i, 0))] + [
        full(a) for a in args[1:]]
    sc = pl.BlockSpec((1, 1), lambda i: (0, 0))
    out = pl.pallas_call(
        _body,
        grid=(NBLK,),
        in_specs=in_specs,
        out_specs=[pl.BlockSpec((BLK, 1), lambda i: (i, 0)),
                   sc, sc, sc, sc, sc],
        out_shape=[jax.ShapeDtypeStruct((NTOK, 1), jnp.int32)] + [
            jax.ShapeDtypeStruct((1, 1), jnp.float32)] * 5,
        scratch_shapes=[pltpu.VMEM((B, CDIM), jnp.float32),
                        pltpu.VMEM((CDIM, K), jnp.float32),
                        pltpu.VMEM((1, K), jnp.float32),
                        pltpu.SMEM((1, 1), jnp.float32),
                        pltpu.SMEM((1, 1), jnp.float32)],
    )(*args)
    idx, recon, commit, cbl, contr, total = out
    return (idx.reshape(B, T), recon[0, 0], commit[0, 0], cbl[0, 0],
            contr[0, 0], total[0, 0])
